# baseline (speedup 1.0000x reference)
import functools

import jax
import jax.numpy as jnp
from jax import lax
from jax.experimental import pallas as pl
from jax.experimental.pallas import tpu as pltpu
from jax.experimental.pallas import tpu_sc as plsc

_HI = jax.lax.Precision.HIGHEST
_LP = 128
_CH = 128


def _e2_body(emb_ref, o_ref):
    E = emb_ref[...]
    ones = jnp.ones((8, E.shape[1]), jnp.float32)
    o_ref[...] = jax.lax.dot_general(
        ones, E * E, (((1,), (1,)), ((), ())),
        precision=_HI, preferred_element_type=jnp.float32)


def _enc_body(x_ref, w_in_ref, b_in_ref, emb_ref, e2_ref, ind_ref):
    X = x_ref[0]
    Wi = w_in_ref[...]
    E = emb_ref[...]

    Z = jax.lax.dot_general(X, Wi, (((0,), (0,)), ((), ())),
                            preferred_element_type=jnp.float32)
    Z = Z + b_in_ref[...]

    x2 = jnp.sum(Z * Z, axis=1, keepdims=True)
    S = jax.lax.dot_general(Z, E, (((1,), (1,)), ((), ())),
                            preferred_element_type=jnp.float32)
    dist = -(x2 - 2.0 * S + e2_ref[...])
    ind_ref[0, 0] = jnp.argmax(dist, axis=1)


def _dec_body(q_ref, w_out_ref, b_out_ref, o_ref):
    Qb = q_ref[0][:, :w_out_ref.shape[0]]
    Wo = w_out_ref[...]
    Y = jax.lax.dot_general(Wo, Qb, (((0,), (1,)), ((), ())),
                            preferred_element_type=jnp.float32)
    o_ref[0] = Y + b_out_ref[...]


def _make_sc_gather(BT):
    info = plsc.get_sparse_core_info()
    NW = info.num_cores * info.num_subcores
    bpw = BT // NW
    n_ch = bpw // _CH
    mesh = plsc.VectorSubcoreMesh(core_axis_name="c", subcore_axis_name="s")

    @functools.partial(
        pl.kernel, mesh=mesh,
        out_type=jax.ShapeDtypeStruct((BT, _LP), jnp.float32),
        scratch_types=[
            pltpu.VMEM((n_ch, _CH), jnp.int32),
            pltpu.VMEM((_CH, _LP), jnp.float32),
            pltpu.VMEM((_CH, _LP), jnp.float32),
            pltpu.SemaphoreType.DMA,
            pltpu.SemaphoreType.DMA,
        ],
    )
    def gather_k(ind_hbm, emb_hbm, out_hbm, idx_v, buf0, buf1, sem0, sem1):
        wid = lax.axis_index("s") * info.num_cores + lax.axis_index("c")
        base = wid * bpw
        pltpu.sync_copy(ind_hbm.at[wid], idx_v)
        bufs = (buf0, buf1)
        sems = (sem0, sem1)
        copies = [None, None]
        for j in range(n_ch):
            p = j % 2
            if copies[p] is not None:
                copies[p].wait()
                pltpu.sync_copy(bufs[p],
                                out_hbm.at[pl.ds(base + (j - 2) * _CH, _CH)])
            copies[p] = pltpu.async_copy(emb_hbm.at[idx_v.at[j]],
                                         bufs[p], sems[p])
        for j in (n_ch - 2, n_ch - 1):
            p = j % 2
            copies[p].wait()
            pltpu.sync_copy(bufs[p], out_hbm.at[pl.ds(base + j * _CH, _CH)])

    return gather_k


def kernel(hidden_states, W_in, b_in, embed, W_out, b_out):
    B, D, T = hidden_states.shape
    K, CD = embed.shape
    Tb = 2048

    b_in2 = b_in.reshape(1, CD)
    b_out2 = b_out.reshape(D, 1)
    emb_pad = jnp.pad(embed, ((0, 0), (0, _LP - CD)))

    e2 = pl.pallas_call(
        _e2_body,
        out_shape=jax.ShapeDtypeStruct((8, K), jnp.float32),
    )(embed)[0:1]

    grid = (B, T // Tb)
    ind = pl.pallas_call(
        _enc_body,
        grid=grid,
        in_specs=[
            pl.BlockSpec((1, D, Tb), lambda b, t: (b, 0, t)),
            pl.BlockSpec((D, CD), lambda b, t: (0, 0)),
            pl.BlockSpec((1, CD), lambda b, t: (0, 0)),
            pl.BlockSpec((K, CD), lambda b, t: (0, 0)),
            pl.BlockSpec((1, K), lambda b, t: (0, 0)),
        ],
        out_specs=pl.BlockSpec((1, 1, Tb), lambda b, t: (b, 0, t)),
        out_shape=jax.ShapeDtypeStruct((B, 1, T), jnp.int32),
    )(hidden_states, W_in, b_in2, embed, e2)

    BT = B * T
    info = plsc.get_sparse_core_info()
    NW = info.num_cores * info.num_subcores
    ind_w = ind.reshape(NW, (BT // NW) // _CH, _CH)
    Q = _make_sc_gather(BT)(ind_w, emb_pad)
    Qb = Q.reshape(B, T, _LP)

    out = pl.pallas_call(
        _dec_body,
        grid=grid,
        in_specs=[
            pl.BlockSpec((1, Tb, _LP), lambda b, t: (b, t, 0)),
            pl.BlockSpec((CD, D), lambda b, t: (0, 0)),
            pl.BlockSpec((D, 1), lambda b, t: (0, 0)),
        ],
        out_specs=pl.BlockSpec((1, D, Tb), lambda b, t: (b, 0, t)),
        out_shape=jax.ShapeDtypeStruct((B, D, T), jnp.float32),
    )(Qb, W_out, b_out2)
    return out


# speedup vs baseline: 2.2404x; 2.2404x over previous
import jax
import jax.numpy as jnp
from jax.experimental import pallas as pl

_HI = jax.lax.Precision.HIGHEST


def _prep_body(emb_ref, e2_ref, eq_ref):
    E = emb_ref[...]
    ones = jnp.ones((8, E.shape[1]), jnp.float32)
    e2_ref[...] = jax.lax.dot_general(
        ones, E * E, (((1,), (1,)), ((), ())),
        precision=_HI, preferred_element_type=jnp.float32)
    eq_ref[...] = jnp.concatenate(
        [E, jnp.ones((E.shape[0], 1), jnp.float32)], axis=1)


def _vq_body(x_ref, w_in_ref, b_in_ref, eq_ref, w_out_ref, b_out_ref,
             e2_ref, o_ref):
    X = x_ref[0]
    Wi = w_in_ref[...]
    Eq = eq_ref[...]
    Wo = w_out_ref[...]
    CD = Wo.shape[0]
    E = Eq[:, :CD]

    Z = jax.lax.dot_general(X, Wi, (((0,), (0,)), ((), ())),
                            preferred_element_type=jnp.float32)
    Z = Z + b_in_ref[...]

    x2 = jnp.sum(Z * Z, axis=1, keepdims=True)
    S = jax.lax.dot_general(Z, E, (((1,), (1,)), ((), ())),
                            preferred_element_type=jnp.float32)
    dist = -(x2 - 2.0 * S + e2_ref[...])

    M = jnp.max(dist, axis=1, keepdims=True)
    oh = (dist == M).astype(jnp.float32)

    Qc = jax.lax.dot_general(Eq, oh, (((0,), (1,)), ((), ())),
                             preferred_element_type=jnp.float32)
    Q = Qc[:CD, :] / Qc[CD:CD + 1, :]
    Y = jax.lax.dot_general(Wo, Q, (((0,), (0,)), ((), ())),
                            preferred_element_type=jnp.float32)
    o_ref[0] = Y + b_out_ref[...]


def kernel(hidden_states, W_in, b_in, embed, W_out, b_out):
    B, D, T = hidden_states.shape
    K, CD = embed.shape
    Tb = 2048

    b_in2 = b_in.reshape(1, CD)
    b_out2 = b_out.reshape(D, 1)

    e2, eq = pl.pallas_call(
        _prep_body,
        out_shape=[jax.ShapeDtypeStruct((8, K), jnp.float32),
                   jax.ShapeDtypeStruct((K, CD + 1), jnp.float32)],
    )(embed)
    e2 = e2[0:1]

    grid = (B, T // Tb)
    out = pl.pallas_call(
        _vq_body,
        grid=grid,
        in_specs=[
            pl.BlockSpec((1, D, Tb), lambda b, t: (b, 0, t)),
            pl.BlockSpec((D, CD), lambda b, t: (0, 0)),
            pl.BlockSpec((1, CD), lambda b, t: (0, 0)),
            pl.BlockSpec((K, CD + 1), lambda b, t: (0, 0)),
            pl.BlockSpec((CD, D), lambda b, t: (0, 0)),
            pl.BlockSpec((D, 1), lambda b, t: (0, 0)),
            pl.BlockSpec((1, K), lambda b, t: (0, 0)),
        ],
        out_specs=pl.BlockSpec((1, D, Tb), lambda b, t: (b, 0, t)),
        out_shape=jax.ShapeDtypeStruct((B, D, T), jnp.float32),
    )(hidden_states, W_in, b_in2, eq, W_out, b_out2, e2)
    return out


# speedup vs baseline: 2.2488x; 1.0037x over previous
import jax
import jax.numpy as jnp
from jax.experimental import pallas as pl

_HI = jax.lax.Precision.HIGHEST


def _e2_body(emb_ref, o_ref):
    E = emb_ref[...]
    ones = jnp.ones((8, E.shape[1]), jnp.float32)
    o_ref[...] = jax.lax.dot_general(
        ones, E * E, (((1,), (1,)), ((), ())),
        precision=_HI, preferred_element_type=jnp.float32)


def _vq_body(x_ref, w_in_ref, b_in_ref, emb_ref, w_out_ref, b_out_ref,
             e2_ref, o_ref):
    X = x_ref[0]
    Wi = w_in_ref[...]
    E = emb_ref[...]
    Wo = w_out_ref[...]

    Z = jax.lax.dot_general(X, Wi, (((0,), (0,)), ((), ())),
                            preferred_element_type=jnp.float32)
    Z = Z + b_in_ref[...]

    x2 = jnp.sum(Z * Z, axis=1, keepdims=True)
    S = jax.lax.dot_general(Z, E, (((1,), (1,)), ((), ())),
                            preferred_element_type=jnp.float32)
    dist = -(x2 - 2.0 * S + e2_ref[...])

    ind = jnp.argmax(dist, axis=1)
    iota = jax.lax.broadcasted_iota(jnp.int32, dist.shape, 1)
    oh = (iota == ind[:, None]).astype(jnp.float32)

    Q = jax.lax.dot_general(E, oh, (((0,), (1,)), ((), ())),
                            preferred_element_type=jnp.float32)
    Y = jax.lax.dot_general(Wo, Q, (((0,), (0,)), ((), ())),
                            preferred_element_type=jnp.float32)
    o_ref[0] = Y + b_out_ref[...]


def kernel(hidden_states, W_in, b_in, embed, W_out, b_out):
    B, D, T = hidden_states.shape
    K, CD = embed.shape
    Tb = 2048

    b_in2 = b_in.reshape(1, CD)
    b_out2 = b_out.reshape(D, 1)

    e2 = pl.pallas_call(
        _e2_body,
        out_shape=jax.ShapeDtypeStruct((8, K), jnp.float32),
    )(embed)[0:1]

    grid = (B, T // Tb)
    out = pl.pallas_call(
        _vq_body,
        grid=grid,
        in_specs=[
            pl.BlockSpec((1, D, Tb), lambda b, t: (b, 0, t)),
            pl.BlockSpec((D, CD), lambda b, t: (0, 0)),
            pl.BlockSpec((1, CD), lambda b, t: (0, 0)),
            pl.BlockSpec((K, CD), lambda b, t: (0, 0)),
            pl.BlockSpec((CD, D), lambda b, t: (0, 0)),
            pl.BlockSpec((D, 1), lambda b, t: (0, 0)),
            pl.BlockSpec((1, K), lambda b, t: (0, 0)),
        ],
        out_specs=pl.BlockSpec((1, D, Tb), lambda b, t: (b, 0, t)),
        out_shape=jax.ShapeDtypeStruct((B, D, T), jnp.float32),
    )(hidden_states, W_in, b_in2, embed, W_out, b_out2, e2)
    return out
